# baseline (speedup 1.0000x reference)
import functools
import numpy as np
import jax
import jax.numpy as jnp
from jax import lax
from jax.experimental import pallas as pl
from jax.experimental.pallas import tpu as pltpu
from jax.experimental.pallas import tpu_sc as plsc

N_BINS = 18
B = 32
T = 16384
NW = 32
TW = T // NW
NBINS_PAD = N_BINS + 2
ACC = NBINS_PAD * B

_INV_DELTA = np.float32(N_BINS / (2.0 * np.pi))
_PI = np.float32(np.pi)


def _binify_body(pha_ref, offs_ref):
    f = (pha_ref[...] + _PI) * _INV_DELTA
    idx = f.astype(jnp.int32)
    idx = jnp.minimum(idx, N_BINS)
    idx = jnp.where(f < 0.0, N_BINS, idx)
    offs_ref[...] = idx * B


def _sc_body(offs_hbm, ampT_hbm, out_hbm, offs_v, amp_v, acc_v, sem):
    w = lax.axis_index("s") * 2 + lax.axis_index("c")

    pltpu.async_copy(
        offs_hbm.at[pl.ds(w * (B * TW), B * TW)], offs_v, sem
    )

    def zero(k, _):
        acc_v[pl.ds(k * 16, 16)] = jnp.zeros((16,), jnp.float32)
        return 0

    lax.fori_loop(0, B * ACC // 16, zero, 0)

    pltpu.make_async_copy(
        offs_hbm.at[pl.ds(0, B * TW)], offs_v, sem
    ).wait()
    pltpu.sync_copy(ampT_hbm.at[pl.ds(w * (TW * B), TW * B)], amp_v)

    lane_iota = lax.iota(jnp.int32, 16)

    def per_j(jj, _):
        obase = jj * TW
        base0 = lane_iota + jj * ACC
        base1 = base0 + 16

        @plsc.parallel_loop(0, TW // 16, unroll=2)
        def _(g):
            off_vec = offs_v[pl.ds(obase + g * 16, 16)]
            tbase = g * (16 * B)
            for u in range(16):
                off_splat = jnp.take_along_axis(
                    off_vec, jnp.full((16,), u, jnp.int32), axis=0,
                    mode="promise_in_bounds"
                )
                a0 = amp_v[pl.ds(tbase + u * B, 16)]
                a1 = amp_v[pl.ds(tbase + u * B + 16, 16)]
                plsc.addupdate_scatter(acc_v, [base0 + off_splat], a0)
                plsc.addupdate_scatter(acc_v, [base1 + off_splat], a1)

        return 0

    lax.fori_loop(0, B, per_j, 0)
    pltpu.sync_copy(acc_v, out_hbm.at[pl.ds(w * (B * ACC), B * ACC)])


@functools.partial(
    pl.kernel,
    out_type=jax.ShapeDtypeStruct((NW * B * ACC,), jnp.float32),
    mesh=plsc.VectorSubcoreMesh(
        core_axis_name="c", subcore_axis_name="s", num_cores=2, num_subcores=16
    ),
    scratch_types=[
        pltpu.VMEM((B * TW,), jnp.int32),
        pltpu.VMEM((TW * B,), jnp.float32),
        pltpu.VMEM((B * ACC,), jnp.float32),
        pltpu.SemaphoreType.DMA,
    ],
    compiler_params=pltpu.CompilerParams(needs_layout_passes=False),
)
def _sc_binsum(offs_hbm, ampT_hbm, out_hbm, offs_v, amp_v, acc_v, sem):
    _sc_body(offs_hbm, ampT_hbm, out_hbm, offs_v, amp_v, acc_v, sem)


def _entropy_body(sums_ref, out_ref):
    s = sums_ref[...]
    r = jnp.sum(s, axis=0)
    s18 = r[:, :N_BINS, :]
    tot = jnp.sum(s18, axis=1, keepdims=True)
    p = s18 / tot
    inv_log_n = np.float32(1.0 / np.log(float(N_BINS)))
    mi = 1.0 + inv_log_n * jnp.sum(p * jnp.log(p), axis=1)
    out_ref[...] = mi


@jax.jit
def kernel(pha, amp):
    offs = pl.pallas_call(
        _binify_body,
        out_shape=jax.ShapeDtypeStruct((B, T), jnp.int32),
    )(pha)
    offs_t = offs.reshape(B, NW, TW).swapaxes(0, 1).reshape(-1)
    sums = _sc_binsum(offs_t, amp.T.reshape(-1))
    mit = pl.pallas_call(
        _entropy_body,
        out_shape=jax.ShapeDtypeStruct((B, B), jnp.float32),
    )(sums.reshape(NW, B, NBINS_PAD, B))
    return mit.T
